# baseline (speedup 1.0000x reference)
import functools

import jax
import jax.numpy as jnp
from jax import lax
from jax.experimental import pallas as pl
from jax.experimental.pallas import tpu as pltpu
from jax.experimental.pallas import tpu_sc as plsc

_NUM_ELEMENTS = 118
_EMB = 64
_NUM_GRAPHS = 64
_FBLOCK = 10000
_LANES = 16
_NBINS = 128
_UNROLL = 1


def _tc_table_body(emb_ref, w1_ref, b1_ref, w2_ref, b2_ref,
                   w3_ref, b3_ref, table_ref):
    h1 = jax.nn.silu(
        jnp.dot(emb_ref[...], w1_ref[...],
                preferred_element_type=jnp.float32) + b1_ref[...])
    h2 = jax.nn.silu(
        jnp.dot(h1, w2_ref[...],
                preferred_element_type=jnp.float32) + b2_ref[...])
    table_ref[...] = (
        jnp.dot(h2, w3_ref[...],
                preferred_element_type=jnp.float32) + b3_ref[...])


def _tc_forces_body(forces_ref):
    forces_ref[...] = jnp.zeros_like(forces_ref)


def _tc_reduce_body(p_ref, e_ref):
    s = jnp.sum(p_ref[...], axis=0, keepdims=True)
    e_ref[...] = s[:, :_NUM_GRAPHS] + s[:, _NUM_GRAPHS:]


def _make_sc_segsum(num_workers, chunk):
    mesh = plsc.VectorSubcoreMesh(core_axis_name="c", subcore_axis_name="s")
    acc_size = _LANES * _NBINS

    @functools.partial(
        pl.kernel,
        mesh=mesh,
        out_type=jax.ShapeDtypeStruct((num_workers * _NUM_GRAPHS,),
                                      jnp.float32),
        compiler_params=pltpu.CompilerParams(needs_layout_passes=False),
        scratch_types=[
            pltpu.VMEM((_NBINS,), jnp.float32),
            pltpu.VMEM((chunk,), jnp.int32),
            pltpu.VMEM((chunk,), jnp.int32),
            pltpu.VMEM((acc_size,), jnp.float32),
            pltpu.VMEM((_NUM_GRAPHS,), jnp.float32),
        ],
    )
    def _sc_segsum(table_hbm, z_hbm, b_hbm, out_hbm,
                   table_v, z_v, b_v, acc_v, e_v):
        wid = lax.axis_index("s") * 2 + lax.axis_index("c")
        base = wid * chunk
        pltpu.sync_copy(table_hbm, table_v)
        pltpu.sync_copy(z_hbm.at[pl.ds(base, chunk)], z_v)
        pltpu.sync_copy(b_hbm.at[pl.ds(base, chunk)], b_v)

        zero16 = jnp.zeros((_LANES,), jnp.float32)

        def _zinit(i, carry):
            acc_v[pl.ds(i * _LANES, _LANES)] = zero16
            return carry

        lax.fori_loop(0, acc_size // _LANES, _zinit, 0)

        lane_off = lax.iota(jnp.int32, _LANES) * _NBINS

        unroll = _UNROLL
        assert chunk % (_LANES * unroll) == 0

        def _accumulate(i, carry):
            base = i * (_LANES * unroll)
            for u in range(unroll):
                zz = z_v[pl.ds(base + u * _LANES, _LANES)]
                bb = b_v[pl.ds(base + u * _LANES, _LANES)]
                val = plsc.load_gather(table_v, [zz])
                plsc.addupdate_scatter(acc_v, [lane_off + bb], val)
            return carry

        lax.fori_loop(0, chunk // (_LANES * unroll), _accumulate, 0)

        for k in range(_NUM_GRAPHS // _LANES):
            s = zero16
            for l in range(_LANES):
                s = s + acc_v[pl.ds(l * _NBINS + k * _LANES, _LANES)]
            e_v[pl.ds(k * _LANES, _LANES)] = s

        pltpu.sync_copy(e_v, out_hbm.at[pl.ds(wid * _NUM_GRAPHS,
                                              _NUM_GRAPHS)])

    return _sc_segsum


def kernel(positions, atomic_numbers, edge_index, batch, embed,
           W1, b1, W2, b2, W3, b3):
    n = atomic_numbers.shape[0]
    nfb = n // _FBLOCK
    assert nfb * _FBLOCK == n

    info = plsc.get_sparse_core_info()
    num_workers = info.num_cores * info.num_subcores
    step = _LANES * _UNROLL
    chunk = -(-n // (num_workers * step)) * step
    n_pad = num_workers * chunk

    z_flat = jnp.pad(atomic_numbers.astype(jnp.int32), (0, n_pad - n))
    b_flat = jnp.pad(batch.astype(jnp.int32), (0, n_pad - n),
                     constant_values=_NUM_GRAPHS)

    emb128 = jnp.zeros((128, _EMB), jnp.float32).at[:_NUM_ELEMENTS, :].set(
        embed.astype(jnp.float32))
    b1r = b1.astype(jnp.float32).reshape(1, -1)
    b2r = b2.astype(jnp.float32).reshape(1, -1)
    b3r = b3.astype(jnp.float32).reshape(1, 1)

    table2d = pl.pallas_call(
        _tc_table_body,
        out_shape=jax.ShapeDtypeStruct((128, 1), jnp.float32),
    )(emb128, W1.astype(jnp.float32), b1r,
      W2.astype(jnp.float32), b2r, W3.astype(jnp.float32), b3r)

    forces = pl.pallas_call(
        _tc_forces_body,
        grid=(nfb,),
        out_specs=pl.BlockSpec((_FBLOCK, 3), lambda i: (i, 0)),
        out_shape=jax.ShapeDtypeStruct((n, 3), jnp.float32),
    )()

    table_flat = table2d.reshape(128)
    partial = _make_sc_segsum(num_workers, chunk)(table_flat, z_flat, b_flat)
    partial2d = partial.reshape(num_workers * _NUM_GRAPHS // 128, 128)

    energy2d = pl.pallas_call(
        _tc_reduce_body,
        out_shape=jax.ShapeDtypeStruct((1, _NUM_GRAPHS), jnp.float32),
    )(partial2d)
    return energy2d.reshape(_NUM_GRAPHS), forces


# speedup vs baseline: 1.9345x; 1.9345x over previous
import functools

import jax
import jax.numpy as jnp
from jax import lax
from jax.experimental import pallas as pl
from jax.experimental.pallas import tpu as pltpu
from jax.experimental.pallas import tpu_sc as plsc

_NUM_ELEMENTS = 118
_EMB = 64
_NUM_GRAPHS = 64
_FBLOCK = 10000
_LANES = 16
_NBINS = 128
_UNROLL = 1


def _tc_table_body(emb_ref, w1_ref, b1_ref, w2_ref, b2_ref,
                   w3_ref, b3_ref, table_ref, fzero_ref):
    h1 = jax.nn.silu(
        jnp.dot(emb_ref[...], w1_ref[...],
                preferred_element_type=jnp.float32) + b1_ref[...])
    h2 = jax.nn.silu(
        jnp.dot(h1, w2_ref[...],
                preferred_element_type=jnp.float32) + b2_ref[...])
    table_ref[...] = (
        jnp.dot(h2, w3_ref[...],
                preferred_element_type=jnp.float32) + b3_ref[...])
    fzero_ref[...] = jnp.zeros_like(fzero_ref)


def _tc_reduce_body(p_ref, e_ref):
    s = jnp.sum(p_ref[...], axis=0, keepdims=True)
    e_ref[...] = s[:, :_NUM_GRAPHS] + s[:, _NUM_GRAPHS:]


def _make_sc_segsum(num_workers, chunk):
    mesh = plsc.VectorSubcoreMesh(core_axis_name="c", subcore_axis_name="s")
    acc_size = _LANES * _NBINS

    @functools.partial(
        pl.kernel,
        mesh=mesh,
        out_type=jax.ShapeDtypeStruct((num_workers * _NUM_GRAPHS,),
                                      jnp.float32),
        compiler_params=pltpu.CompilerParams(needs_layout_passes=False),
        scratch_types=[
            pltpu.VMEM((_NBINS,), jnp.float32),
            pltpu.VMEM((chunk,), jnp.int32),
            pltpu.VMEM((chunk,), jnp.int32),
            pltpu.VMEM((acc_size,), jnp.float32),
            pltpu.VMEM((_NUM_GRAPHS,), jnp.float32),
        ],
    )
    def _sc_segsum(table_hbm, z_hbm, b_hbm, out_hbm,
                   table_v, z_v, b_v, acc_v, e_v):
        wid = lax.axis_index("s") * 2 + lax.axis_index("c")
        base = wid * chunk
        pltpu.sync_copy(table_hbm, table_v)
        pltpu.sync_copy(z_hbm.at[pl.ds(base, chunk)], z_v)
        pltpu.sync_copy(b_hbm.at[pl.ds(base, chunk)], b_v)

        zero16 = jnp.zeros((_LANES,), jnp.float32)

        def _zinit(i, carry):
            acc_v[pl.ds(i * _LANES, _LANES)] = zero16
            return carry

        lax.fori_loop(0, acc_size // _LANES, _zinit, 0)

        lane_off = lax.iota(jnp.int32, _LANES) * _NBINS

        unroll = _UNROLL
        assert chunk % (_LANES * unroll) == 0

        def _accumulate(i, carry):
            base = i * (_LANES * unroll)
            for u in range(unroll):
                zz = z_v[pl.ds(base + u * _LANES, _LANES)]
                bb = b_v[pl.ds(base + u * _LANES, _LANES)]
                val = plsc.load_gather(table_v, [zz])
                plsc.addupdate_scatter(acc_v, [lane_off + bb], val)
            return carry

        lax.fori_loop(0, chunk // (_LANES * unroll), _accumulate, 0)

        for k in range(_NUM_GRAPHS // _LANES):
            s = zero16
            for l in range(_LANES):
                s = s + acc_v[pl.ds(l * _NBINS + k * _LANES, _LANES)]
            e_v[pl.ds(k * _LANES, _LANES)] = s

        pltpu.sync_copy(e_v, out_hbm.at[pl.ds(wid * _NUM_GRAPHS,
                                              _NUM_GRAPHS)])

    return _sc_segsum


def kernel(positions, atomic_numbers, edge_index, batch, embed,
           W1, b1, W2, b2, W3, b3):
    n = atomic_numbers.shape[0]
    info = plsc.get_sparse_core_info()
    num_workers = info.num_cores * info.num_subcores
    step = _LANES * _UNROLL
    chunk = -(-n // (num_workers * step)) * step
    n_pad = num_workers * chunk

    z_flat = jnp.pad(atomic_numbers.astype(jnp.int32), (0, n_pad - n))
    b_flat = jnp.pad(batch.astype(jnp.int32), (0, n_pad - n),
                     constant_values=_NUM_GRAPHS)

    emb128 = jnp.zeros((128, _EMB), jnp.float32).at[:_NUM_ELEMENTS, :].set(
        embed.astype(jnp.float32))
    b1r = b1.astype(jnp.float32).reshape(1, -1)
    b2r = b2.astype(jnp.float32).reshape(1, -1)
    b3r = b3.astype(jnp.float32).reshape(1, 1)

    table2d, fzero = pl.pallas_call(
        _tc_table_body,
        out_shape=[
            jax.ShapeDtypeStruct((128, 1), jnp.float32),
            jax.ShapeDtypeStruct((1, 1), jnp.float32),
        ],
    )(emb128, W1.astype(jnp.float32), b1r,
      W2.astype(jnp.float32), b2r, W3.astype(jnp.float32), b3r)

    forces = jnp.broadcast_to(fzero.reshape(()), (n, 3))

    table_flat = table2d.reshape(128)
    partial = _make_sc_segsum(num_workers, chunk)(table_flat, z_flat, b_flat)
    partial2d = partial.reshape(num_workers * _NUM_GRAPHS // 128, 128)

    energy2d = pl.pallas_call(
        _tc_reduce_body,
        out_shape=jax.ShapeDtypeStruct((1, _NUM_GRAPHS), jnp.float32),
    )(partial2d)
    return energy2d.reshape(_NUM_GRAPHS), forces
